# baseline (speedup 1.0000x reference)
import functools

import jax
import jax.numpy as jnp
from jax.experimental import pallas as pl
from jax.experimental.pallas import tpu as pltpu

NUM_CLASSES = 2
THRESHOLD = 0.35
NEGPOS_RATIO = 3
VAR0 = 0.1
VAR1 = 0.2
SMOOTH_POINT = 0.2

LANES = 128


def _smooth_l1(d):
    return jnp.where(d < 1.0, 0.5 * d * d, d - 0.5)


def _mbl_kernel(loc_ref, conf_ref, iou_ref, pri_ref, tgt_ref,
                bbox_ref, iouh_ref, lm_ref, cls_ref, npos_ref,
                *, T, P, R):
    f32 = jnp.float32
    shape = (R, LANES)
    lin = (jax.lax.broadcasted_iota(jnp.int32, shape, 0) * LANES
           + jax.lax.broadcasted_iota(jnp.int32, shape, 1))
    valid = lin < P

    pcx = pri_ref[0]
    pcy = pri_ref[1]
    pw = pri_ref[2]
    ph = pri_ref[3]
    px1 = pcx - pw * 0.5
    py1 = pcy - ph * 0.5
    px2 = pcx + pw * 0.5
    py2 = pcy + ph * 0.5
    area_p = pw * ph

    tr = [[tgt_ref[0, t, f] for f in range(14)] for f2 in range(1) for t in range(T)]

    col_max = jnp.full(shape, -1.0, f32)
    col_arg = jnp.zeros(shape, jnp.int32)
    for t in range(T):
        tx1, ty1, tx2, ty2 = tr[t][0], tr[t][1], tr[t][2], tr[t][3]
        area_t = (tx2 - tx1) * (ty2 - ty1)
        iw = jnp.maximum(jnp.minimum(px2, tx2) - jnp.maximum(px1, tx1), 0.0)
        ih = jnp.maximum(jnp.minimum(py2, ty2) - jnp.maximum(py1, ty1), 0.0)
        inter = iw * ih
        ov = inter / (area_t + area_p - inter)
        mv = jnp.max(ov)
        upd = ov > col_max
        col_max = jnp.where(upd, ov, col_max)
        col_arg = jnp.where(upd, t, col_arg)
        mf = ov == mv
        col_max = jnp.where(mf, 2.0, col_max)
        col_arg = jnp.where(mf, t, col_arg)

    pos = col_max >= THRESHOLD
    num_pos = jnp.sum(jnp.where(pos, 1.0, 0.0))

    nbits = max(1, (T - 1).bit_length())
    bits = [(col_arg & (1 << b)) != 0 for b in range(nbits)]

    def gather32(vals):
        def rec(base, level):
            if level < 0:
                return vals[base]
            lo = rec(base, level - 1)
            idx = base + (1 << level)
            hi = rec(idx, level - 1) if idx < T else lo
            return jnp.where(bits[level], hi, lo)
        return rec(0, nbits - 1)

    locT = jnp.transpose(loc_ref[0], (1, 0))
    locP = jnp.concatenate(
        [locT, jnp.zeros((14, R * LANES - locT.shape[1]), f32)], axis=1)
    loc3 = locP.reshape(14, R, LANES)
    l = [loc3[f] for f in range(14)]
    p1x = VAR0 * l[0]
    p1y = VAR0 * l[1]
    p2x = p1x + jnp.exp(VAR1 * l[2])
    p2y = p1y + jnp.exp(VAR1 * l[3])
    m0 = gather32([tr[t][0] for t in range(T)])
    m1 = gather32([tr[t][1] for t in range(T)])
    m2 = gather32([tr[t][2] for t in range(T)])
    m3 = gather32([tr[t][3] for t in range(T)])
    g1x = ((m0 + m2) * 0.5 - pcx) / pw
    g1y = ((m1 + m3) * 0.5 - pcy) / ph
    g2x = g1x + (m2 - m0) / pw
    g2y = g1y + (m3 - m1) / ph
    ex1 = jnp.minimum(p1x, g1x)
    ey1 = jnp.minimum(p1y, g1y)
    ix1 = jnp.maximum(p1x, g1x)
    iy1 = jnp.maximum(p1y, g1y)
    ix2 = jnp.minimum(p2x, g2x)
    iy2 = jnp.minimum(p2y, g2y)
    xmin = jnp.minimum(ix1, ix2)
    ymin = jnp.minimum(iy1, iy2)
    xmax = jnp.maximum(ix1, ix2)
    ymax = jnp.maximum(iy1, iy2)
    inter = ((ix2 - ex1) * (iy2 - ey1) + (xmin - ex1) * (ymin - ey1)
             - (ix1 - ex1) * (ymax - ey1) - (xmax - ex1) * (iy1 - ey1))
    union = (p2x - p1x) * (p2y - p1y) + (g2x - g1x) * (g2y - g1y) - inter
    iou = inter / jnp.maximum(union, 1e-10)
    ss = jnp.where(iou < SMOOTH_POINT, 1.0, 0.0)
    one_m = 1.0 - iou
    eiou = (0.5 / SMOOTH_POINT) * ss * one_m * one_m + (1.0 - ss) * (
        one_m - 0.5 * SMOOTH_POINT)
    bbox_sum = jnp.sum(jnp.where(pos, eiou, 0.0))

    lm_acc = jnp.zeros(shape, f32)
    inv_vw = 1.0 / (VAR0 * pw)
    inv_vh = 1.0 / (VAR0 * ph)
    for a in range(5):
        gx = (gather32([tr[t][4 + 2 * a] for t in range(T)]) - pcx) * inv_vw
        gy = (gather32([tr[t][5 + 2 * a] for t in range(T)]) - pcy) * inv_vh
        lm_acc = lm_acc + _smooth_l1(jnp.abs(l[4 + 2 * a] - gx))
        lm_acc = lm_acc + _smooth_l1(jnp.abs(l[5 + 2 * a] - gy))
    lm_sum = jnp.sum(jnp.where(pos, lm_acc, 0.0))

    iouh_sum = jnp.sum(jnp.where(
        pos, _smooth_l1(jnp.abs(iou_ref[0] - col_max)), 0.0))

    c0 = conf_ref[0, 0]
    c1 = conf_ref[0, 1]
    mx = jnp.maximum(c0, c1)
    lse = mx + jnp.log(jnp.exp(c0 - mx) + jnp.exp(c1 - mx))
    nll_pos = jnp.sum(jnp.where(pos, lse - c1, 0.0))
    loss_c = jnp.where(jnp.logical_or(pos, jnp.logical_not(valid)),
                       0.0, lse - c0)

    k = jnp.minimum(jnp.sum(jnp.where(pos, 1, 0)) * NEGPOS_RATIO, P - 1)
    maxv = jnp.max(loss_c)
    hi0 = jax.lax.bitcast_convert_type(maxv, jnp.int32)

    def body(_, carry):
        lo, hi = carry
        mid = lo + (hi - lo) // 2
        tau = jax.lax.bitcast_convert_type(mid, f32)
        cnt = jnp.sum(jnp.where(loss_c > tau, 1, 0))
        pred = cnt < k
        return (jnp.where(pred, lo, mid + 1), jnp.where(pred, mid, hi))

    lo, hi = jax.lax.fori_loop(0, 31, body, (jnp.int32(0), hi0))
    tau = jax.lax.bitcast_convert_type(hi, f32)
    gt = loss_c > tau
    cnt_gt = jnp.sum(jnp.where(gt, 1, 0))
    sum_gt = jnp.sum(jnp.where(gt, loss_c, 0.0))
    topk = sum_gt + (k - cnt_gt).astype(f32) * tau

    bbox_ref[0, 0, 0] = bbox_sum
    iouh_ref[0, 0, 0] = iouh_sum
    lm_ref[0, 0, 0] = lm_sum
    cls_ref[0, 0, 0] = nll_pos + topk
    npos_ref[0, 0, 0] = num_pos


def kernel(loc_data, conf_data, iou_data, priors, targets):
    num, P, _ = loc_data.shape
    T = targets.shape[1]
    PP = ((P + LANES - 1) // LANES) * LANES
    R = PP // LANES
    pad = PP - P

    loc_t = loc_data
    conf_t = jnp.pad(jnp.transpose(conf_data, (0, 2, 1)),
                     ((0, 0), (0, 0), (0, pad))).reshape(num, NUM_CLASSES, R, LANES)
    iou_t = jnp.pad(iou_data[:, :, 0], ((0, 0), (0, pad))).reshape(num, R, LANES)
    pri_t = jnp.pad(jnp.transpose(priors, (1, 0)),
                    ((0, 0), (0, pad))).reshape(4, R, LANES)

    out_shape = [jax.ShapeDtypeStruct((num, 1, 1), jnp.float32)] * 5
    body = functools.partial(_mbl_kernel, T=T, P=P, R=R)
    outs = pl.pallas_call(
        body,
        grid=(num,),
        in_specs=[
            pl.BlockSpec((1, P, 14), lambda i: (i, 0, 0)),
            pl.BlockSpec((1, NUM_CLASSES, R, LANES), lambda i: (i, 0, 0, 0)),
            pl.BlockSpec((1, R, LANES), lambda i: (i, 0, 0)),
            pl.BlockSpec((4, R, LANES), lambda i: (0, 0, 0)),
            pl.BlockSpec((1, T, 15), lambda i: (i, 0, 0)),
        ],
        out_specs=[pl.BlockSpec((1, 1, 1), lambda i: (i, 0, 0),
                                memory_space=pltpu.SMEM)] * 5,
        out_shape=out_shape,
        compiler_params=pltpu.CompilerParams(
            dimension_semantics=("parallel",)),
    )(loc_t, conf_t, iou_t, pri_t, targets)

    bbox, iouh, lm, cls, npos = [jnp.sum(o) for o in outs]
    N = jnp.maximum(npos, 1.0)
    return (bbox / N, iouh / N, lm / (N * 5.0), cls / N)


# speedup vs baseline: 1.7738x; 1.7738x over previous
import functools

import jax
import jax.numpy as jnp
from jax.experimental import pallas as pl
from jax.experimental.pallas import tpu as pltpu

NUM_CLASSES = 2
THRESHOLD = 0.35
NEGPOS_RATIO = 3
VAR0 = 0.1
VAR1 = 0.2
SMOOTH_POINT = 0.2

LANES = 128


def _smooth_l1(d):
    return jnp.where(d < 1.0, 0.5 * d * d, d - 0.5)


def _mbl_kernel(loc_ref, conf_ref, iou_ref, pri_ref, tgt_ref,
                bbox_ref, iouh_ref, lm_ref, nllp_ref, npos_ref, lossc_ref,
                *, T, P, R):
    f32 = jnp.float32
    shape = (R, LANES)

    pcx = pri_ref[0]
    pcy = pri_ref[1]
    pw = pri_ref[2]
    ph = pri_ref[3]
    px1 = pcx - pw * 0.5
    py1 = pcy - ph * 0.5
    px2 = pcx + pw * 0.5
    py2 = pcy + ph * 0.5
    area_p = pw * ph

    tr = [[tgt_ref[0, t, f] for f in range(14)] for f2 in range(1) for t in range(T)]

    col_max = jnp.full(shape, -1.0, f32)
    col_arg = jnp.zeros(shape, jnp.int32)
    for t in range(T):
        tx1, ty1, tx2, ty2 = tr[t][0], tr[t][1], tr[t][2], tr[t][3]
        area_t = (tx2 - tx1) * (ty2 - ty1)
        iw = jnp.maximum(jnp.minimum(px2, tx2) - jnp.maximum(px1, tx1), 0.0)
        ih = jnp.maximum(jnp.minimum(py2, ty2) - jnp.maximum(py1, ty1), 0.0)
        inter = iw * ih
        ov = inter / (area_t + area_p - inter)
        mv = jnp.max(ov)
        upd = ov > col_max
        col_max = jnp.where(upd, ov, col_max)
        col_arg = jnp.where(upd, t, col_arg)
        mf = ov == mv
        col_max = jnp.where(mf, 2.0, col_max)
        col_arg = jnp.where(mf, t, col_arg)

    pos = col_max >= THRESHOLD
    num_pos = jnp.sum(jnp.where(pos, 1.0, 0.0))

    nbits = max(1, (T - 1).bit_length())
    bits = [(col_arg & (1 << b)) != 0 for b in range(nbits)]

    def gather32(vals):
        def rec(base, level):
            if level < 0:
                return vals[base]
            lo = rec(base, level - 1)
            idx = base + (1 << level)
            hi = rec(idx, level - 1) if idx < T else lo
            return jnp.where(bits[level], hi, lo)
        return rec(0, nbits - 1)

    l = [loc_ref[0, f] for f in range(14)]
    p1x = VAR0 * l[0]
    p1y = VAR0 * l[1]
    p2x = p1x + jnp.exp(VAR1 * l[2])
    p2y = p1y + jnp.exp(VAR1 * l[3])
    m0 = gather32([tr[t][0] for t in range(T)])
    m1 = gather32([tr[t][1] for t in range(T)])
    m2 = gather32([tr[t][2] for t in range(T)])
    m3 = gather32([tr[t][3] for t in range(T)])
    g1x = ((m0 + m2) * 0.5 - pcx) / pw
    g1y = ((m1 + m3) * 0.5 - pcy) / ph
    g2x = g1x + (m2 - m0) / pw
    g2y = g1y + (m3 - m1) / ph
    ex1 = jnp.minimum(p1x, g1x)
    ey1 = jnp.minimum(p1y, g1y)
    ix1 = jnp.maximum(p1x, g1x)
    iy1 = jnp.maximum(p1y, g1y)
    ix2 = jnp.minimum(p2x, g2x)
    iy2 = jnp.minimum(p2y, g2y)
    xmin = jnp.minimum(ix1, ix2)
    ymin = jnp.minimum(iy1, iy2)
    xmax = jnp.maximum(ix1, ix2)
    ymax = jnp.maximum(iy1, iy2)
    inter = ((ix2 - ex1) * (iy2 - ey1) + (xmin - ex1) * (ymin - ey1)
             - (ix1 - ex1) * (ymax - ey1) - (xmax - ex1) * (iy1 - ey1))
    union = (p2x - p1x) * (p2y - p1y) + (g2x - g1x) * (g2y - g1y) - inter
    iou = inter / jnp.maximum(union, 1e-10)
    ss = jnp.where(iou < SMOOTH_POINT, 1.0, 0.0)
    one_m = 1.0 - iou
    eiou = (0.5 / SMOOTH_POINT) * ss * one_m * one_m + (1.0 - ss) * (
        one_m - 0.5 * SMOOTH_POINT)
    bbox_sum = jnp.sum(jnp.where(pos, eiou, 0.0))

    lm_acc = jnp.zeros(shape, f32)
    inv_vw = 1.0 / (VAR0 * pw)
    inv_vh = 1.0 / (VAR0 * ph)
    for a in range(5):
        gx = (gather32([tr[t][4 + 2 * a] for t in range(T)]) - pcx) * inv_vw
        gy = (gather32([tr[t][5 + 2 * a] for t in range(T)]) - pcy) * inv_vh
        lm_acc = lm_acc + _smooth_l1(jnp.abs(l[4 + 2 * a] - gx))
        lm_acc = lm_acc + _smooth_l1(jnp.abs(l[5 + 2 * a] - gy))
    lm_sum = jnp.sum(jnp.where(pos, lm_acc, 0.0))

    iouh_sum = jnp.sum(jnp.where(
        pos, _smooth_l1(jnp.abs(iou_ref[0] - col_max)), 0.0))

    c0 = conf_ref[0, 0]
    c1 = conf_ref[0, 1]
    mx = jnp.maximum(c0, c1)
    lse = mx + jnp.log(jnp.exp(c0 - mx) + jnp.exp(c1 - mx))
    nll_pos = jnp.sum(jnp.where(pos, lse - c1, 0.0))
    lossc_ref[0] = jnp.where(pos, 0.0, lse - c0)

    bbox_ref[0, 0, 0] = bbox_sum
    iouh_ref[0, 0, 0] = iouh_sum
    lm_ref[0, 0, 0] = lm_sum
    nllp_ref[0, 0, 0] = nll_pos
    npos_ref[0, 0, 0] = num_pos


def _topk_kernel(lossc_ref, npos_ref, topk_ref, *, P):
    f32 = jnp.float32
    v = lossc_ref[...]
    k = jnp.minimum(npos_ref[...].astype(jnp.int32) * NEGPOS_RATIO, P - 1)
    maxv = jnp.max(v, axis=(1, 2), keepdims=True)
    hi0 = jax.lax.bitcast_convert_type(maxv, jnp.int32)
    lo0 = jnp.zeros_like(hi0)

    def body(_, carry):
        lo, hi = carry
        mid = lo + (hi - lo) // 2
        tau = jax.lax.bitcast_convert_type(mid, f32)
        cnt = jnp.sum(jnp.where(v > tau, 1, 0), axis=(1, 2), keepdims=True)
        pred = cnt < k
        return (jnp.where(pred, lo, mid + 1), jnp.where(pred, mid, hi))

    lo, hi = jax.lax.fori_loop(0, 31, body, (lo0, hi0))
    tau = jax.lax.bitcast_convert_type(hi, f32)
    gt = v > tau
    cnt_gt = jnp.sum(jnp.where(gt, 1, 0), axis=(1, 2), keepdims=True)
    sum_gt = jnp.sum(jnp.where(gt, v, 0.0), axis=(1, 2), keepdims=True)
    topk_ref[...] = sum_gt + (k - cnt_gt).astype(f32) * tau


def kernel(loc_data, conf_data, iou_data, priors, targets):
    num, P, _ = loc_data.shape
    T = targets.shape[1]
    PP = ((P + LANES - 1) // LANES) * LANES
    R = PP // LANES
    pad = PP - P

    loc_t = jnp.pad(jnp.transpose(loc_data, (0, 2, 1)),
                    ((0, 0), (0, 0), (0, pad))).reshape(num, 14, R, LANES)
    conf_pad = jnp.concatenate(
        [jnp.zeros((num, 1, pad), jnp.float32),
         jnp.full((num, 1, pad), -1e30, jnp.float32)], axis=1)
    conf_t = jnp.concatenate(
        [jnp.transpose(conf_data, (0, 2, 1)), conf_pad],
        axis=2).reshape(num, NUM_CLASSES, R, LANES)
    iou_t = jnp.pad(iou_data[:, :, 0], ((0, 0), (0, pad))).reshape(num, R, LANES)
    pri_t = jnp.pad(jnp.transpose(priors, (1, 0)),
                    ((0, 0), (0, pad))).reshape(4, R, LANES)

    out_shape = ([jax.ShapeDtypeStruct((num, 1, 1), jnp.float32)] * 5
                 + [jax.ShapeDtypeStruct((num, R, LANES), jnp.float32)])
    body = functools.partial(_mbl_kernel, T=T, P=P, R=R)
    outs = pl.pallas_call(
        body,
        grid=(num,),
        in_specs=[
            pl.BlockSpec((1, 14, R, LANES), lambda i: (i, 0, 0, 0)),
            pl.BlockSpec((1, NUM_CLASSES, R, LANES), lambda i: (i, 0, 0, 0)),
            pl.BlockSpec((1, R, LANES), lambda i: (i, 0, 0)),
            pl.BlockSpec((4, R, LANES), lambda i: (0, 0, 0)),
            pl.BlockSpec((1, T, 15), lambda i: (i, 0, 0)),
        ],
        out_specs=([pl.BlockSpec((1, 1, 1), lambda i: (i, 0, 0),
                                 memory_space=pltpu.SMEM)] * 5
                   + [pl.BlockSpec((1, R, LANES), lambda i: (i, 0, 0))]),
        out_shape=out_shape,
        compiler_params=pltpu.CompilerParams(
            dimension_semantics=("parallel",)),
    )(loc_t, conf_t, iou_t, pri_t, targets)

    bbox, iouh, lm, nllp, npos = [jnp.sum(o) for o in outs[:5]]
    lossc, npos_per = outs[5], outs[4]

    topk = pl.pallas_call(
        functools.partial(_topk_kernel, P=P),
        in_specs=[
            pl.BlockSpec((num, R, LANES), lambda: (0, 0, 0)),
            pl.BlockSpec((num, 1, 1), lambda: (0, 0, 0)),
        ],
        out_specs=pl.BlockSpec((num, 1, 1), lambda: (0, 0, 0)),
        out_shape=jax.ShapeDtypeStruct((num, 1, 1), jnp.float32),
    )(lossc, npos_per)

    cls = nllp + jnp.sum(topk)
    N = jnp.maximum(npos, 1.0)
    return (bbox / N, iouh / N, lm / (N * 5.0), cls / N)


# speedup vs baseline: 1.7957x; 1.0123x over previous
import functools

import jax
import jax.numpy as jnp
from jax.experimental import pallas as pl
from jax.experimental.pallas import tpu as pltpu

NUM_CLASSES = 2
THRESHOLD = 0.35
NEGPOS_RATIO = 3
VAR0 = 0.1
VAR1 = 0.2
SMOOTH_POINT = 0.2

LANES = 128


def _smooth_l1(d):
    return jnp.where(d < 1.0, 0.5 * d * d, d - 0.5)


def _mbl_kernel(loc_ref, conf_ref, iou_ref, pri_ref, tgt_ref,
                bbox_ref, iouh_ref, lm_ref, nllp_ref, npos_ref, topk_ref,
                lossc_scr, npos_scr,
                *, T, P, R, num):
    f32 = jnp.float32
    shape = (R, LANES)
    i = pl.program_id(0)

    pcx = pri_ref[0]
    pcy = pri_ref[1]
    pw = pri_ref[2]
    ph = pri_ref[3]
    px1 = pcx - pw * 0.5
    py1 = pcy - ph * 0.5
    px2 = pcx + pw * 0.5
    py2 = pcy + ph * 0.5
    area_p = pw * ph

    tr = [[tgt_ref[0, t, f] for f in range(14)] for f2 in range(1) for t in range(T)]

    col_max = jnp.full(shape, -1.0, f32)
    col_arg = jnp.zeros(shape, jnp.int32)
    for t in range(T):
        tx1, ty1, tx2, ty2 = tr[t][0], tr[t][1], tr[t][2], tr[t][3]
        area_t = (tx2 - tx1) * (ty2 - ty1)
        iw = jnp.maximum(jnp.minimum(px2, tx2) - jnp.maximum(px1, tx1), 0.0)
        ih = jnp.maximum(jnp.minimum(py2, ty2) - jnp.maximum(py1, ty1), 0.0)
        inter = iw * ih
        ov = inter / (area_t + area_p - inter)
        mv = jnp.max(ov)
        upd = ov > col_max
        col_max = jnp.where(upd, ov, col_max)
        col_arg = jnp.where(upd, t, col_arg)
        mf = ov == mv
        col_max = jnp.where(mf, 2.0, col_max)
        col_arg = jnp.where(mf, t, col_arg)

    pos = col_max >= THRESHOLD
    num_pos = jnp.sum(jnp.where(pos, 1.0, 0.0))

    nbits = max(1, (T - 1).bit_length())
    bits = [(col_arg & (1 << b)) != 0 for b in range(nbits)]

    def gather32(vals):
        def rec(base, level):
            if level < 0:
                return vals[base]
            lo = rec(base, level - 1)
            idx = base + (1 << level)
            hi = rec(idx, level - 1) if idx < T else lo
            return jnp.where(bits[level], hi, lo)
        return rec(0, nbits - 1)

    l = [loc_ref[0, f] for f in range(14)]
    p1x = VAR0 * l[0]
    p1y = VAR0 * l[1]
    p2x = p1x + jnp.exp(VAR1 * l[2])
    p2y = p1y + jnp.exp(VAR1 * l[3])
    m0 = gather32([tr[t][0] for t in range(T)])
    m1 = gather32([tr[t][1] for t in range(T)])
    m2 = gather32([tr[t][2] for t in range(T)])
    m3 = gather32([tr[t][3] for t in range(T)])
    g1x = ((m0 + m2) * 0.5 - pcx) / pw
    g1y = ((m1 + m3) * 0.5 - pcy) / ph
    g2x = g1x + (m2 - m0) / pw
    g2y = g1y + (m3 - m1) / ph
    ex1 = jnp.minimum(p1x, g1x)
    ey1 = jnp.minimum(p1y, g1y)
    ix1 = jnp.maximum(p1x, g1x)
    iy1 = jnp.maximum(p1y, g1y)
    ix2 = jnp.minimum(p2x, g2x)
    iy2 = jnp.minimum(p2y, g2y)
    xmin = jnp.minimum(ix1, ix2)
    ymin = jnp.minimum(iy1, iy2)
    xmax = jnp.maximum(ix1, ix2)
    ymax = jnp.maximum(iy1, iy2)
    inter = ((ix2 - ex1) * (iy2 - ey1) + (xmin - ex1) * (ymin - ey1)
             - (ix1 - ex1) * (ymax - ey1) - (xmax - ex1) * (iy1 - ey1))
    union = (p2x - p1x) * (p2y - p1y) + (g2x - g1x) * (g2y - g1y) - inter
    iou = inter / jnp.maximum(union, 1e-10)
    ss = jnp.where(iou < SMOOTH_POINT, 1.0, 0.0)
    one_m = 1.0 - iou
    eiou = (0.5 / SMOOTH_POINT) * ss * one_m * one_m + (1.0 - ss) * (
        one_m - 0.5 * SMOOTH_POINT)
    bbox_sum = jnp.sum(jnp.where(pos, eiou, 0.0))

    lm_acc = jnp.zeros(shape, f32)
    inv_vw = 1.0 / (VAR0 * pw)
    inv_vh = 1.0 / (VAR0 * ph)
    for a in range(5):
        gx = (gather32([tr[t][4 + 2 * a] for t in range(T)]) - pcx) * inv_vw
        gy = (gather32([tr[t][5 + 2 * a] for t in range(T)]) - pcy) * inv_vh
        lm_acc = lm_acc + _smooth_l1(jnp.abs(l[4 + 2 * a] - gx))
        lm_acc = lm_acc + _smooth_l1(jnp.abs(l[5 + 2 * a] - gy))
    lm_sum = jnp.sum(jnp.where(pos, lm_acc, 0.0))

    iouh_sum = jnp.sum(jnp.where(
        pos, _smooth_l1(jnp.abs(iou_ref[0] - col_max)), 0.0))

    c0 = conf_ref[0, 0]
    c1 = conf_ref[0, 1]
    mx = jnp.maximum(c0, c1)
    lse = mx + jnp.log(jnp.exp(c0 - mx) + jnp.exp(c1 - mx))
    nll_pos = jnp.sum(jnp.where(pos, lse - c1, 0.0))
    lossc_scr[i] = jnp.where(pos, 0.0, lse - c0)
    npos_scr[i] = num_pos

    bbox_ref[0, 0, 0] = bbox_sum
    iouh_ref[0, 0, 0] = iouh_sum
    lm_ref[0, 0, 0] = lm_sum
    nllp_ref[0, 0, 0] = nll_pos
    npos_ref[0, 0, 0] = num_pos

    @pl.when(i == num - 1)
    def _topk():
        v = lossc_scr[...]
        npv = jnp.stack([npos_scr[j] for j in range(num)]).reshape(num, 1, 1)
        k = jnp.minimum(npv.astype(jnp.int32) * NEGPOS_RATIO, P - 1)
        maxv = jnp.max(v, axis=(1, 2), keepdims=True)
        hi0 = jax.lax.bitcast_convert_type(maxv, jnp.int32)
        lo0 = jnp.zeros_like(hi0)

        def body(_, carry):
            lo, hi = carry
            mid = lo + (hi - lo) // 2
            tau = jax.lax.bitcast_convert_type(mid, f32)
            cnt = jnp.sum(jnp.where(v > tau, 1, 0), axis=(1, 2),
                          keepdims=True)
            pred = cnt < k
            return (jnp.where(pred, lo, mid + 1), jnp.where(pred, mid, hi))

        lo, hi = jax.lax.fori_loop(0, 31, body, (lo0, hi0))
        tau = jax.lax.bitcast_convert_type(hi, f32)
        gt = v > tau
        cnt_gt = jnp.sum(jnp.where(gt, 1, 0), axis=(1, 2), keepdims=True)
        sum_gt = jnp.sum(jnp.where(gt, v, 0.0), axis=(1, 2), keepdims=True)
        topk_ref[...] = sum_gt + (k - cnt_gt).astype(f32) * tau


def kernel(loc_data, conf_data, iou_data, priors, targets):
    num, P, _ = loc_data.shape
    T = targets.shape[1]
    PP = ((P + LANES - 1) // LANES) * LANES
    R = PP // LANES
    pad = PP - P

    loc_t = jnp.pad(jnp.transpose(loc_data, (0, 2, 1)),
                    ((0, 0), (0, 0), (0, pad))).reshape(num, 14, R, LANES)
    conf_pad = jnp.concatenate(
        [jnp.zeros((num, 1, pad), jnp.float32),
         jnp.full((num, 1, pad), -1e30, jnp.float32)], axis=1)
    conf_t = jnp.concatenate(
        [jnp.transpose(conf_data, (0, 2, 1)), conf_pad],
        axis=2).reshape(num, NUM_CLASSES, R, LANES)
    iou_t = jnp.pad(iou_data[:, :, 0], ((0, 0), (0, pad))).reshape(num, R, LANES)
    pri_t = jnp.pad(jnp.transpose(priors, (1, 0)),
                    ((0, 0), (0, pad))).reshape(4, R, LANES)

    out_shape = [jax.ShapeDtypeStruct((num, 1, 1), jnp.float32)] * 6
    body = functools.partial(_mbl_kernel, T=T, P=P, R=R, num=num)
    outs = pl.pallas_call(
        body,
        grid=(num,),
        in_specs=[
            pl.BlockSpec((1, 14, R, LANES), lambda i: (i, 0, 0, 0)),
            pl.BlockSpec((1, NUM_CLASSES, R, LANES), lambda i: (i, 0, 0, 0)),
            pl.BlockSpec((1, R, LANES), lambda i: (i, 0, 0)),
            pl.BlockSpec((4, R, LANES), lambda i: (0, 0, 0)),
            pl.BlockSpec((1, T, 15), lambda i: (i, 0, 0)),
        ],
        out_specs=([pl.BlockSpec((1, 1, 1), lambda i: (i, 0, 0),
                                 memory_space=pltpu.SMEM)] * 5
                   + [pl.BlockSpec((num, 1, 1), lambda i: (0, 0, 0))]),
        out_shape=out_shape,
        scratch_shapes=[
            pltpu.VMEM((num, R, LANES), jnp.float32),
            pltpu.SMEM((num,), jnp.float32),
        ],
    )(loc_t, conf_t, iou_t, pri_t, targets)

    bbox, iouh, lm, nllp, npos = [jnp.sum(o) for o in outs[:5]]
    cls = nllp + jnp.sum(outs[5])
    N = jnp.maximum(npos, 1.0)
    return (bbox / N, iouh / N, lm / (N * 5.0), cls / N)
